# baseline (speedup 1.0000x reference)
import functools

import jax
import jax.numpy as jnp
from jax import lax
from jax.experimental import pallas as pl
from jax.experimental.pallas import tpu as pltpu
from jax.experimental.pallas import tpu_sc as plsc

N = 10000
E = 320000
HID = 128
HEADS = 8
BASES = 4
NAGG = 3
NG = 400
F = HID // HEADS
BF = BASES * F
VOCAB = 28

NC = 2
NS = 16
NW = NC * NS
NPT = 320
NPAD = NW * NPT

CH = 128
CAP = E + CH
CHE = 3200
GPC = CHE // 16
NCHE = E // CHE
S = 512
STG = S + 160
PACK_SHIFT = 14

def _wid():
    return lax.axis_index("s") * NC + lax.axis_index("c")


def _prep_body(src_hbm, dst_hbm, pd_hbm, cnt_hbm, sbuf, dbuf, stage, cbuf,
               ss0, sd0, ss1, sd1):
    w = _wid()
    lo = w * NPT
    hi = lo + NPT

    def group(args, carry):
        half, g = args
        wptr, off = carry
        sl = pl.ds(half * CHE + g * 16, 16)
        dv = dbuf[sl]
        sv = sbuf[sl]
        mask = (dv >= lo) & (dv < hi)
        pk = sv | ((dv - lo) << PACK_SHIFT)
        plsc.store_compressed(stage.at[pl.ds(wptr, 16)], pk, mask=mask)
        wptr = wptr + plsc.all_reduce_population_count(mask)[0]
        flush = wptr >= S

        @pl.when(flush)
        def _():
            pltpu.sync_copy(stage.at[pl.ds(0, S)],
                            pd_hbm.at[pl.ds(pl.multiple_of(w * CAP + off, 8), S)])
            tail = stage[pl.ds(S, 16)]
            stage[pl.ds(0, 16)] = tail

        wptr = jnp.where(flush, wptr - S, wptr)
        off = jnp.where(flush, off + S, off)
        return wptr, off

    def g2a(i, carry):
        return group((0, i * 2 + 1), group((0, i * 2), carry))

    def g2b(i, carry):
        return group((1, i * 2 + 1), group((1, i * 2), carry))

    sb = (sbuf.at[pl.ds(0, CHE)], sbuf.at[pl.ds(CHE, CHE)])
    db = (dbuf.at[pl.ds(0, CHE)], dbuf.at[pl.ds(CHE, CHE)])
    sems = ((ss0, sd0), (ss1, sd1))

    def issue(b, ci):
        cic = jnp.minimum(ci, NCHE - 1) * CHE
        pltpu.async_copy(src_hbm.at[pl.ds(pl.multiple_of(cic, 8), CHE)],
                         sb[b], sems[b][0])
        pltpu.async_copy(dst_hbm.at[pl.ds(pl.multiple_of(cic, 8), CHE)],
                         db[b], sems[b][1])

    def wait(b):
        pltpu.make_async_copy(src_hbm.at[pl.ds(0, CHE)], sb[b], sems[b][0]).wait()
        pltpu.make_async_copy(dst_hbm.at[pl.ds(0, CHE)], db[b], sems[b][1]).wait()

    issue(0, 0)

    def pair(p, carry):
        issue(1, 2 * p + 1)
        wait(0)
        carry = lax.fori_loop(0, GPC // 2, g2a, carry)
        issue(0, 2 * p + 2)
        wait(1)
        return lax.fori_loop(0, GPC // 2, g2b, carry)

    wptr, off = lax.fori_loop(0, NCHE // 2, pair,
                              (jnp.int32(0), jnp.int32(0)))
    wait(0)

    cnt = off + wptr
    padv = jnp.full((16,), NPT << PACK_SHIFT, jnp.int32)
    iota16 = lax.iota(jnp.int32, 16)
    for i in range(CH // 16):
        plsc.store_scatter(stage, [wptr + i * 16 + iota16], padv)
    pad_len = (-cnt) % CH
    nflush = (wptr + pad_len) // CH

    def flush_tail(j, _):
        pltpu.sync_copy(stage.at[pl.ds(j * CH, CH)],
                        pd_hbm.at[pl.ds(pl.multiple_of(w * CAP + off + j * CH, 8), CH)])
        return 0

    lax.fori_loop(0, nflush, flush_tail, 0)
    cbuf[pl.ds(0, 16)] = jnp.full((16,), cnt, jnp.int32)
    pltpu.sync_copy(cbuf.at[pl.ds(0, 8)], cnt_hbm.at[pl.ds(pl.multiple_of(w * 8, 8), 8)])


@functools.cache
def _prep_fn():
    mesh = plsc.VectorSubcoreMesh(core_axis_name="c", subcore_axis_name="s",
                                  num_cores=NC, num_subcores=NS)
    return pl.kernel(
        _prep_body,
        out_type=(
            jax.ShapeDtypeStruct((NW * CAP,), jnp.int32),
            jax.ShapeDtypeStruct((NW * 8,), jnp.int32),
        ),
        mesh=mesh,
        compiler_params=pltpu.CompilerParams(needs_layout_passes=False,
                                             disable_bounds_checks=True),
        scratch_types=[
            pltpu.VMEM((2 * CHE,), jnp.int32),
            pltpu.VMEM((2 * CHE,), jnp.int32),
            pltpu.VMEM((STG,), jnp.int32),
            pltpu.VMEM((16,), jnp.int32),
            pltpu.SemaphoreType.DMA,
            pltpu.SemaphoreType.DMA,
            pltpu.SemaphoreType.DMA,
            pltpu.SemaphoreType.DMA,
        ],
    )


def _agg_body(bases_hbm, pd_hbm, cnt_hbm, asum_hbm, amax_hbm, deg_hbm,
              cbuf, pdbuf0, pdbuf1, idx0, idx1, ldst0, ldst1, mbuf0, mbuf1,
              s0, s1, s2, s3, m0, m1, m2, m3, acc_x, deg_l,
              spd0, spd1, sg0, sg1):
    w = _wid()
    accs = (s0, s1, s2, s3)
    accm = (m0, m1, m2, m3)
    zf = jnp.zeros((16,), jnp.float32)
    ninf = jnp.full((16,), -3.4e38, jnp.float32)

    def zrow(r, _):
        for f in range(BF // 16):
            accs[f][pl.ds(r * 16, 16)] = zf
            accm[f][pl.ds(r * 16, 16)] = ninf
        return 0

    lax.fori_loop(0, NPT + 16, zrow, 0)
    for i in range((NPT + 16) // 16):
        deg_l[pl.ds(i * 16, 16)] = zf

    pltpu.sync_copy(cnt_hbm.at[pl.ds(pl.multiple_of(w * 8, 8), 8)], cbuf.at[pl.ds(0, 8)])
    cnt = cbuf[pl.ds(0, 16)][0]
    nch = (cnt + CH - 1) // CH
    ones16 = jnp.ones((16,), jnp.float32)

    pdb = (pdbuf0, pdbuf1)
    idxb = (idx0, idx1)
    ldb = (ldst0, ldst1)
    mb = (mbuf0, mbuf1)
    spd = (spd0, spd1)
    sg = (sg0, sg1)

    def issue_pd(b, ci):
        cic = jnp.minimum(ci, jnp.maximum(nch - 1, 0)) * CH
        pltpu.async_copy(
            pd_hbm.at[pl.ds(pl.multiple_of(w * CAP + cic, 8), CH)],
            pdb[b], spd[b])

    def wait_pd(b):
        pltpu.make_async_copy(pd_hbm.at[pl.ds(0, CH)], pdb[b], spd[b]).wait()

    def unpack_gather(b):
        for g in range(CH // 16):
            sl = pl.ds(g * 16, 16)
            pv = pdb[b][sl]
            idxb[b][sl] = pv & ((1 << PACK_SHIFT) - 1)
            ldb[b][sl] = pv >> PACK_SHIFT
        pltpu.async_copy(bases_hbm.at[idxb[b]], mb[b], sg[b])

    def wait_gather(b):
        pltpu.make_async_copy(bases_hbm.at[idxb[b]], mb[b], sg[b]).wait()

    def process(b):
        def egroup(g, __):
            d_vec = ldb[b][pl.ds(g * 16, 16)]
            plsc.addupdate_scatter(deg_l, [d_vec], ones16)
            for j in range(16):
                dsl = pl.ds(d_vec[j] * 16, 16)
                for f in range(BF // 16):
                    m = mb[b][g * 16 + j, pl.ds(f * 16, 16)]
                    accs[f][dsl] = accs[f][dsl] + m
                    accm[f][dsl] = jnp.maximum(accm[f][dsl], m)
            return __

        lax.fori_loop(0, CH // 16, egroup, 0)

    issue_pd(0, 0)
    wait_pd(0)

    @pl.when(nch > 0)
    def _():
        unpack_gather(0)

    issue_pd(1, 1)

    def pair(p, _):
        wait_pd(1)
        unpack_gather(1)
        issue_pd(0, 2 * p + 2)
        wait_gather(0)
        process(0)
        wait_pd(0)
        unpack_gather(0)
        issue_pd(1, 2 * p + 3)
        wait_gather(1)

        @pl.when(2 * p + 1 < nch)
        def _():
            process(1)

        return 0

    lax.fori_loop(0, (nch + 1) // 2, pair, 0)
    wait_pd(1)

    @pl.when(nch > 0)
    def _():
        wait_gather(0)

    def ileave(acc4):
        def row(r, _):
            for f in range(BF // 16):
                acc_x[r, pl.ds(f * 16, 16)] = acc4[f][pl.ds(r * 16, 16)]
            return 0
        lax.fori_loop(0, NPT, row, 0)

    ileave(accs)
    pltpu.sync_copy(acc_x.at[pl.ds(0, NPT)], asum_hbm.at[w])
    ileave(accm)
    pltpu.sync_copy(acc_x.at[pl.ds(0, NPT)], amax_hbm.at[w])
    pltpu.sync_copy(deg_l.at[pl.ds(0, NPT)], deg_hbm.at[pl.ds(pl.multiple_of(w * NPT, 8), NPT)])


@functools.cache
def _agg_fn():
    mesh = plsc.VectorSubcoreMesh(core_axis_name="c", subcore_axis_name="s",
                                  num_cores=NC, num_subcores=NS)
    return pl.kernel(
        _agg_body,
        out_type=(
            jax.ShapeDtypeStruct((NW, NPT, BF), jnp.float32),
            jax.ShapeDtypeStruct((NW, NPT, BF), jnp.float32),
            jax.ShapeDtypeStruct((NW * NPT,), jnp.float32),
        ),
        mesh=mesh,
        compiler_params=pltpu.CompilerParams(needs_layout_passes=False,
                                             disable_bounds_checks=True),
        scratch_types=[
            pltpu.VMEM((16,), jnp.int32),
            pltpu.VMEM((CH,), jnp.int32),
            pltpu.VMEM((CH,), jnp.int32),
            pltpu.VMEM((CH,), jnp.int32),
            pltpu.VMEM((CH,), jnp.int32),
            pltpu.VMEM((CH,), jnp.int32),
            pltpu.VMEM((CH,), jnp.int32),
            pltpu.VMEM((CH, HID), jnp.float32),
            pltpu.VMEM((CH, HID), jnp.float32),
            pltpu.VMEM(((NPT + 16) * 16,), jnp.float32),
            pltpu.VMEM(((NPT + 16) * 16,), jnp.float32),
            pltpu.VMEM(((NPT + 16) * 16,), jnp.float32),
            pltpu.VMEM(((NPT + 16) * 16,), jnp.float32),
            pltpu.VMEM(((NPT + 16) * 16,), jnp.float32),
            pltpu.VMEM(((NPT + 16) * 16,), jnp.float32),
            pltpu.VMEM(((NPT + 16) * 16,), jnp.float32),
            pltpu.VMEM(((NPT + 16) * 16,), jnp.float32),
            pltpu.VMEM((NPT + 16, BF), jnp.float32),
            pltpu.VMEM((NPT + 16,), jnp.float32),
            pltpu.SemaphoreType.DMA,
            pltpu.SemaphoreType.DMA,
            pltpu.SemaphoreType.DMA,
            pltpu.SemaphoreType.DMA,
        ],
    )


def _tcA0_body(x_ref, emb_ref, wb_ref, wcp_ref, bcp_ref,
               h_ref, bases_ref, w2_ref):
    oh = (x_ref[...] == lax.broadcasted_iota(jnp.int32, (N, 32), 1))
    h = jnp.dot(oh.astype(jnp.float32), emb_ref[...],
                preferred_element_type=jnp.float32)
    h_ref[...] = h
    b = jnp.dot(h, wb_ref[...], preferred_element_type=jnp.float32)
    bases_ref[...] = jnp.concatenate([b, jnp.zeros((N, HID - BF), jnp.float32)],
                                     axis=1)
    w2_ref[...] = jnp.dot(h, wcp_ref[...],
                          preferred_element_type=jnp.float32) + bcp_ref[...]


_tcA0 = pl.pallas_call(
    _tcA0_body,
    out_shape=(
        jax.ShapeDtypeStruct((N, HID), jnp.float32),
        jax.ShapeDtypeStruct((N, HID), jnp.float32),
        jax.ShapeDtypeStruct((N, 96), jnp.float32),
    ),
)


def _tcA_body(h_ref, wb_ref, wcp_ref, bcp_ref, bases_ref, w2_ref):
    h = h_ref[...]
    b = jnp.dot(h, wb_ref[...], preferred_element_type=jnp.float32)
    bases_ref[...] = jnp.concatenate([b, jnp.zeros((N, HID - BF), jnp.float32)],
                                     axis=1)
    w2_ref[...] = jnp.dot(h, wcp_ref[...],
                          preferred_element_type=jnp.float32) + bcp_ref[...]


_tcA = pl.pallas_call(
    _tcA_body,
    out_shape=(
        jax.ShapeDtypeStruct((N, HID), jnp.float32),
        jax.ShapeDtypeStruct((N, 96), jnp.float32),
    ),
)


def _tcB_body(w2_ref, asum_ref, amax_ref, deg_ref, bias_ref, gamma_ref,
              beta_ref, ident_ref, out_ref):
    asum = asum_ref[...]
    deg = deg_ref[...]
    rdeg = 1.0 / jnp.maximum(deg, 1.0)
    amean = asum * rdeg
    amax = jnp.where(deg > 0.0, amax_ref[...], 0.0)
    w2 = w2_ref[...]
    col8 = lax.broadcasted_iota(jnp.int32, (8, HID), 1)
    row8 = lax.broadcasted_iota(jnp.int32, (8, HID), 0)
    e8 = (col8 // F == row8).astype(jnp.float32)
    col16 = lax.broadcasted_iota(jnp.int32, (F, HID), 1)
    row16 = lax.broadcasted_iota(jnp.int32, (F, HID), 0)
    t16 = (col16 % F == row16).astype(jnp.float32)
    aggs = (asum, amean, amax)
    out = jnp.zeros((N, HID), jnp.float32)
    for a in range(NAGG * BASES):
        k, b = divmod(a, BASES)
        wa = w2[:, a * 8:(a + 1) * 8]
        va = aggs[k][:, b * F:(b + 1) * F]
        out = out + (jnp.dot(wa, e8, preferred_element_type=jnp.float32) *
                     jnp.dot(va, t16, preferred_element_type=jnp.float32))
    out = out + bias_ref[...]
    mu = jnp.mean(out, axis=0, keepdims=True)
    var = jnp.mean((out - mu) ** 2, axis=0, keepdims=True)
    out = (out - mu) * lax.rsqrt(var + 1e-5) * gamma_ref[...] + beta_ref[...]
    out_ref[...] = jnp.maximum(out, 0.0) + ident_ref[...]


_tcB = pl.pallas_call(
    _tcB_body,
    out_shape=jax.ShapeDtypeStruct((N, HID), jnp.float32),
)


def _tcF_body(h_ref, bat_ref, w1_ref, b1_ref, w2_ref, b2_ref, w3_ref, b3_ref,
              y_ref):
    oht = (lax.broadcasted_iota(jnp.int32, (NG, N), 0) == bat_ref[...])
    oht = oht.astype(jnp.float32)
    gsum = jnp.dot(oht, h_ref[...], preferred_element_type=jnp.float32)
    cnt = jnp.sum(oht, axis=1, keepdims=True)
    g = gsum / jnp.maximum(cnt, 1.0)
    y = jnp.maximum(jnp.dot(g, w1_ref[...],
                            preferred_element_type=jnp.float32) + b1_ref[...], 0.0)
    y = jnp.maximum(jnp.dot(y, w2_ref[...],
                            preferred_element_type=jnp.float32) + b2_ref[...], 0.0)
    y_ref[...] = jnp.dot(y, w3_ref[...],
                         preferred_element_type=jnp.float32) + b3_ref[...]


_tcF = pl.pallas_call(
    _tcF_body,
    out_shape=jax.ShapeDtypeStruct((NG, 8), jnp.float32),
)


def kernel(x, edge_index, batch, params):
    src = edge_index[0]
    dst = edge_index[1]
    pd, cntm = _prep_fn()(src, dst)

    emb = jnp.zeros((32, HID), jnp.float32).at[:VOCAB].set(params['emb'])
    x2d = x.reshape(N, 1)

    h = None
    deg = None
    for i, lp in enumerate(params['layers']):
        wcp = lp['Wc'].reshape(HID, HEADS, NAGG * BASES).transpose(0, 2, 1)
        wcp = wcp.reshape(HID, HEADS * NAGG * BASES)
        bcp = lp['bc'].reshape(HEADS, NAGG * BASES).T.reshape(1, -1)
        if i == 0:
            h, bases, w2 = _tcA0(x2d, emb, lp['Wb'], wcp, bcp)
        else:
            bases, w2 = _tcA(h, lp['Wb'], wcp, bcp)
        asum3, amax3, deg3 = _agg_fn()(bases, pd, cntm)
        asum = asum3.reshape(NPAD, BF)[:N]
        amax = amax3.reshape(NPAD, BF)[:N]
        if deg is None:
            deg = deg3.reshape(NPAD, 1)[:N]
        h = _tcB(w2, asum, amax, deg, lp['bias'].reshape(1, HID),
                 lp['gamma'].reshape(1, HID), lp['beta'].reshape(1, HID), h)

    mp = params['mlp']
    w3 = jnp.zeros((HID // 4, 8), jnp.float32).at[:, :1].set(mp['W3'])
    b3 = jnp.zeros((1, 8), jnp.float32).at[:, :1].set(mp['b3'].reshape(1, 1))
    y8 = _tcF(h, batch.reshape(1, N), mp['W1'], mp['b1'].reshape(1, -1),
              mp['W2'], mp['b2'].reshape(1, -1), w3, b3)
    return y8[:, :1]


# speedup vs baseline: 1.0138x; 1.0138x over previous
import functools

import jax
import jax.numpy as jnp
from jax import lax
from jax.experimental import pallas as pl
from jax.experimental.pallas import tpu as pltpu
from jax.experimental.pallas import tpu_sc as plsc

N = 10000
E = 320000
HID = 128
HEADS = 8
BASES = 4
NAGG = 3
NG = 400
F = HID // HEADS
BF = BASES * F
VOCAB = 28

NC = 2
NS = 16
NW = NC * NS
NPT = 320
NPAD = NW * NPT

CH = 128
CAP = E + CH
CHE = 3200
GPC = CHE // 16
NCHE = E // CHE
S = 512
STG = S + 160
PACK_SHIFT = 14

def _wid():
    return lax.axis_index("s") * NC + lax.axis_index("c")


def _prep_body(src_hbm, dst_hbm, pd_hbm, cnt_hbm, sbuf, dbuf, stage, cbuf,
               ss0, sd0, ss1, sd1):
    w = _wid()
    lo = w * NPT
    hi = lo + NPT

    def group(args, carry):
        half, g = args
        wptr, off = carry
        sl = pl.ds(half * CHE + g * 16, 16)
        dv = dbuf[sl]
        sv = sbuf[sl]
        mask = (dv >= lo) & (dv < hi)
        pk = sv | ((dv - lo) << PACK_SHIFT)
        plsc.store_compressed(stage.at[pl.ds(wptr, 16)], pk, mask=mask)
        wptr = wptr + plsc.all_reduce_population_count(mask)[0]
        flush = wptr >= S

        @pl.when(flush)
        def _():
            pltpu.sync_copy(stage.at[pl.ds(0, S)],
                            pd_hbm.at[pl.ds(pl.multiple_of(w * CAP + off, 8), S)])
            tail = stage[pl.ds(S, 16)]
            stage[pl.ds(0, 16)] = tail

        wptr = jnp.where(flush, wptr - S, wptr)
        off = jnp.where(flush, off + S, off)
        return wptr, off

    def g2a(i, carry):
        return group((0, i * 2 + 1), group((0, i * 2), carry))

    def g2b(i, carry):
        return group((1, i * 2 + 1), group((1, i * 2), carry))

    sb = (sbuf.at[pl.ds(0, CHE)], sbuf.at[pl.ds(CHE, CHE)])
    db = (dbuf.at[pl.ds(0, CHE)], dbuf.at[pl.ds(CHE, CHE)])
    sems = ((ss0, sd0), (ss1, sd1))

    def issue(b, ci):
        cic = jnp.minimum(ci, NCHE - 1) * CHE
        pltpu.async_copy(src_hbm.at[pl.ds(pl.multiple_of(cic, 8), CHE)],
                         sb[b], sems[b][0])
        pltpu.async_copy(dst_hbm.at[pl.ds(pl.multiple_of(cic, 8), CHE)],
                         db[b], sems[b][1])

    def wait(b):
        pltpu.make_async_copy(src_hbm.at[pl.ds(0, CHE)], sb[b], sems[b][0]).wait()
        pltpu.make_async_copy(dst_hbm.at[pl.ds(0, CHE)], db[b], sems[b][1]).wait()

    issue(0, 0)

    def pair(p, carry):
        issue(1, 2 * p + 1)
        wait(0)
        carry = lax.fori_loop(0, GPC // 2, g2a, carry)
        issue(0, 2 * p + 2)
        wait(1)
        return lax.fori_loop(0, GPC // 2, g2b, carry)

    wptr, off = lax.fori_loop(0, NCHE // 2, pair,
                              (jnp.int32(0), jnp.int32(0)))
    wait(0)

    cnt = off + wptr
    padv = jnp.full((16,), NPT << PACK_SHIFT, jnp.int32)
    iota16 = lax.iota(jnp.int32, 16)
    for i in range(CH // 16):
        plsc.store_scatter(stage, [wptr + i * 16 + iota16], padv)
    pad_len = (-cnt) % CH
    nflush = (wptr + pad_len) // CH

    def flush_tail(j, _):
        pltpu.sync_copy(stage.at[pl.ds(j * CH, CH)],
                        pd_hbm.at[pl.ds(pl.multiple_of(w * CAP + off + j * CH, 8), CH)])
        return 0

    lax.fori_loop(0, nflush, flush_tail, 0)
    cbuf[pl.ds(0, 16)] = jnp.full((16,), cnt, jnp.int32)
    pltpu.sync_copy(cbuf.at[pl.ds(0, 8)], cnt_hbm.at[pl.ds(pl.multiple_of(w * 8, 8), 8)])


@functools.cache
def _prep_fn():
    mesh = plsc.VectorSubcoreMesh(core_axis_name="c", subcore_axis_name="s",
                                  num_cores=NC, num_subcores=NS)
    return pl.kernel(
        _prep_body,
        out_type=(
            jax.ShapeDtypeStruct((NW * CAP,), jnp.int32),
            jax.ShapeDtypeStruct((NW * 8,), jnp.int32),
        ),
        mesh=mesh,
        compiler_params=pltpu.CompilerParams(needs_layout_passes=False,
                                             disable_bounds_checks=True),
        scratch_types=[
            pltpu.VMEM((2 * CHE,), jnp.int32),
            pltpu.VMEM((2 * CHE,), jnp.int32),
            pltpu.VMEM((STG,), jnp.int32),
            pltpu.VMEM((16,), jnp.int32),
            pltpu.SemaphoreType.DMA,
            pltpu.SemaphoreType.DMA,
            pltpu.SemaphoreType.DMA,
            pltpu.SemaphoreType.DMA,
        ],
    )


def _agg_body(bases_hbm, pd_hbm, cnt_hbm, asum_hbm, amax_hbm, deg_hbm,
              cbuf, pdbuf0, pdbuf1, idx0, idx1, ldst0, ldst1, mbuf0, mbuf1,
              s0, s1, s2, s3, m0, m1, m2, m3, acc_x, deg_l,
              spd0, spd1, sg0, sg1):
    w = _wid()
    accs = (s0, s1, s2, s3)
    accm = (m0, m1, m2, m3)
    zf = jnp.zeros((16,), jnp.float32)
    ninf = jnp.full((16,), -3.4e38, jnp.float32)

    def zrow(r, _):
        for f in range(BF // 16):
            accs[f][pl.ds(r * 16, 16)] = zf
            accm[f][pl.ds(r * 16, 16)] = ninf
        return 0

    lax.fori_loop(0, NPT + 16, zrow, 0)
    for i in range((NPT + 16) // 16):
        deg_l[pl.ds(i * 16, 16)] = zf

    pltpu.sync_copy(cnt_hbm.at[pl.ds(pl.multiple_of(w * 8, 8), 8)], cbuf.at[pl.ds(0, 8)])
    cnt = cbuf[pl.ds(0, 16)][0]
    nch = (cnt + CH - 1) // CH
    ones16 = jnp.ones((16,), jnp.float32)

    pdb = (pdbuf0, pdbuf1)
    idxb = (idx0, idx1)
    ldb = (ldst0, ldst1)
    mb = (mbuf0, mbuf1)
    spd = (spd0, spd1)
    sg = (sg0, sg1)

    def issue_pd(b, ci):
        cic = jnp.minimum(ci, jnp.maximum(nch - 1, 0)) * CH
        pltpu.async_copy(
            pd_hbm.at[pl.ds(pl.multiple_of(w * CAP + cic, 8), CH)],
            pdb[b], spd[b])

    def wait_pd(b):
        pltpu.make_async_copy(pd_hbm.at[pl.ds(0, CH)], pdb[b], spd[b]).wait()

    def unpack_gather(b):
        for g in range(CH // 16):
            sl = pl.ds(g * 16, 16)
            pv = pdb[b][sl]
            idxb[b][sl] = pv & ((1 << PACK_SHIFT) - 1)
            ldb[b][sl] = pv >> PACK_SHIFT
        pltpu.async_copy(bases_hbm.at[idxb[b]], mb[b], sg[b])

    def wait_gather(b):
        pltpu.make_async_copy(bases_hbm.at[idxb[b]], mb[b], sg[b]).wait()

    def process(b):
        def egroup(g, __):
            d_vec = ldb[b][pl.ds(g * 16, 16)]
            plsc.addupdate_scatter(deg_l, [d_vec], ones16)
            for j in range(16):
                dsl = pl.ds(d_vec[j] * 16, 16)
                for f in range(BF // 16):
                    m = mb[b][g * 16 + j, pl.ds(f * 16, 16)]
                    accs[f][dsl] = accs[f][dsl] + m
                    accm[f][dsl] = jnp.maximum(accm[f][dsl], m)
            return __

        lax.fori_loop(0, CH // 16, egroup, 0)

    issue_pd(0, 0)
    wait_pd(0)

    @pl.when(nch > 0)
    def _():
        unpack_gather(0)

    issue_pd(1, 1)

    def pair(p, _):
        wait_pd(1)
        unpack_gather(1)
        issue_pd(0, 2 * p + 2)
        wait_gather(0)
        process(0)
        wait_pd(0)
        unpack_gather(0)
        issue_pd(1, 2 * p + 3)
        wait_gather(1)

        @pl.when(2 * p + 1 < nch)
        def _():
            process(1)

        return 0

    lax.fori_loop(0, (nch + 1) // 2, pair, 0)
    wait_pd(1)

    @pl.when(nch > 0)
    def _():
        wait_gather(0)

    def ileave(acc4):
        def row(r, _):
            for f in range(BF // 16):
                acc_x[r, pl.ds(f * 16, 16)] = acc4[f][pl.ds(r * 16, 16)]
            return 0
        lax.fori_loop(0, NPT, row, 0)

    ileave(accs)
    pltpu.sync_copy(acc_x.at[pl.ds(0, NPT)], asum_hbm.at[w])
    ileave(accm)
    pltpu.sync_copy(acc_x.at[pl.ds(0, NPT)], amax_hbm.at[w])
    pltpu.sync_copy(deg_l.at[pl.ds(0, NPT)], deg_hbm.at[pl.ds(pl.multiple_of(w * NPT, 8), NPT)])


@functools.cache
def _agg_fn():
    mesh = plsc.VectorSubcoreMesh(core_axis_name="c", subcore_axis_name="s",
                                  num_cores=NC, num_subcores=NS)
    return pl.kernel(
        _agg_body,
        out_type=(
            jax.ShapeDtypeStruct((NW, NPT, BF), jnp.float32),
            jax.ShapeDtypeStruct((NW, NPT, BF), jnp.float32),
            jax.ShapeDtypeStruct((NW * NPT,), jnp.float32),
        ),
        mesh=mesh,
        compiler_params=pltpu.CompilerParams(needs_layout_passes=False,
                                             disable_bounds_checks=True),
        scratch_types=[
            pltpu.VMEM((16,), jnp.int32),
            pltpu.VMEM((CH,), jnp.int32),
            pltpu.VMEM((CH,), jnp.int32),
            pltpu.VMEM((CH,), jnp.int32),
            pltpu.VMEM((CH,), jnp.int32),
            pltpu.VMEM((CH,), jnp.int32),
            pltpu.VMEM((CH,), jnp.int32),
            pltpu.VMEM((CH, HID), jnp.float32),
            pltpu.VMEM((CH, HID), jnp.float32),
            pltpu.VMEM(((NPT + 16) * 16,), jnp.float32),
            pltpu.VMEM(((NPT + 16) * 16,), jnp.float32),
            pltpu.VMEM(((NPT + 16) * 16,), jnp.float32),
            pltpu.VMEM(((NPT + 16) * 16,), jnp.float32),
            pltpu.VMEM(((NPT + 16) * 16,), jnp.float32),
            pltpu.VMEM(((NPT + 16) * 16,), jnp.float32),
            pltpu.VMEM(((NPT + 16) * 16,), jnp.float32),
            pltpu.VMEM(((NPT + 16) * 16,), jnp.float32),
            pltpu.VMEM((NPT + 16, BF), jnp.float32),
            pltpu.VMEM((NPT + 16,), jnp.float32),
            pltpu.SemaphoreType.DMA,
            pltpu.SemaphoreType.DMA,
            pltpu.SemaphoreType.DMA,
            pltpu.SemaphoreType.DMA,
        ],
    )


def _tcA0_body(x_ref, emb_ref, wb_ref, wcp_ref, bcp_ref,
               h_ref, bases_ref, w2_ref):
    oh = (x_ref[...] == lax.broadcasted_iota(jnp.int32, (N, 32), 1))
    h = jnp.dot(oh.astype(jnp.float32), emb_ref[...],
                preferred_element_type=jnp.float32)
    h_ref[...] = h
    b = jnp.dot(h, wb_ref[...], preferred_element_type=jnp.float32)
    bases_ref[...] = jnp.concatenate([b, jnp.zeros((N, HID - BF), jnp.float32)],
                                     axis=1)
    w2_ref[...] = jnp.dot(h, wcp_ref[...],
                          preferred_element_type=jnp.float32) + bcp_ref[...]


_tcA0 = pl.pallas_call(
    _tcA0_body,
    out_shape=(
        jax.ShapeDtypeStruct((N, HID), jnp.float32),
        jax.ShapeDtypeStruct((N, HID), jnp.float32),
        jax.ShapeDtypeStruct((N, 96), jnp.float32),
    ),
)


def _tcA_body(h_ref, wb_ref, wcp_ref, bcp_ref, bases_ref, w2_ref):
    h = h_ref[...]
    b = jnp.dot(h, wb_ref[...], preferred_element_type=jnp.float32)
    bases_ref[...] = jnp.concatenate([b, jnp.zeros((N, HID - BF), jnp.float32)],
                                     axis=1)
    w2_ref[...] = jnp.dot(h, wcp_ref[...],
                          preferred_element_type=jnp.float32) + bcp_ref[...]


_tcA = pl.pallas_call(
    _tcA_body,
    out_shape=(
        jax.ShapeDtypeStruct((N, HID), jnp.float32),
        jax.ShapeDtypeStruct((N, 96), jnp.float32),
    ),
)


def _tcB_body(w2_ref, asum_ref, amax_ref, deg_ref, bias_ref, gamma_ref,
              beta_ref, ident_ref, out_ref):
    asum = asum_ref[...][:N]
    deg = deg_ref[...][:N]
    rdeg = 1.0 / jnp.maximum(deg, 1.0)
    amean = asum * rdeg
    amax = jnp.where(deg > 0.0, amax_ref[...][:N], 0.0)
    w2 = w2_ref[...]
    col8 = lax.broadcasted_iota(jnp.int32, (8, HID), 1)
    row8 = lax.broadcasted_iota(jnp.int32, (8, HID), 0)
    e8 = (col8 // F == row8).astype(jnp.float32)
    col16 = lax.broadcasted_iota(jnp.int32, (F, HID), 1)
    row16 = lax.broadcasted_iota(jnp.int32, (F, HID), 0)
    t16 = (col16 % F == row16).astype(jnp.float32)
    aggs = (asum, amean, amax)
    out = jnp.zeros((N, HID), jnp.float32)
    for a in range(NAGG * BASES):
        k, b = divmod(a, BASES)
        wa = w2[:, a * 8:(a + 1) * 8]
        va = aggs[k][:, b * F:(b + 1) * F]
        out = out + (jnp.dot(wa, e8, preferred_element_type=jnp.float32) *
                     jnp.dot(va, t16, preferred_element_type=jnp.float32))
    out = out + bias_ref[...]
    mu = jnp.mean(out, axis=0, keepdims=True)
    var = jnp.mean((out - mu) ** 2, axis=0, keepdims=True)
    out = (out - mu) * lax.rsqrt(var + 1e-5) * gamma_ref[...] + beta_ref[...]
    out_ref[...] = jnp.maximum(out, 0.0) + ident_ref[...]


_tcB = pl.pallas_call(
    _tcB_body,
    out_shape=jax.ShapeDtypeStruct((N, HID), jnp.float32),
)


def _tcF_body(h_ref, bat_ref, w1_ref, b1_ref, w2_ref, b2_ref, w3_ref, b3_ref,
              y_ref):
    oht = (lax.broadcasted_iota(jnp.int32, (NG, N), 0) == bat_ref[...])
    oht = oht.astype(jnp.float32)
    gsum = jnp.dot(oht, h_ref[...], preferred_element_type=jnp.float32)
    cnt = jnp.sum(oht, axis=1, keepdims=True)
    g = gsum / jnp.maximum(cnt, 1.0)
    y = jnp.maximum(jnp.dot(g, w1_ref[...],
                            preferred_element_type=jnp.float32) + b1_ref[...], 0.0)
    y = jnp.maximum(jnp.dot(y, w2_ref[...],
                            preferred_element_type=jnp.float32) + b2_ref[...], 0.0)
    y_ref[...] = jnp.dot(y, w3_ref[...],
                         preferred_element_type=jnp.float32) + b3_ref[...]


_tcF = pl.pallas_call(
    _tcF_body,
    out_shape=jax.ShapeDtypeStruct((NG, 8), jnp.float32),
)


def kernel(x, edge_index, batch, params):
    src = edge_index[0]
    dst = edge_index[1]
    pd, cntm = _prep_fn()(src, dst)

    emb = jnp.zeros((32, HID), jnp.float32).at[:VOCAB].set(params['emb'])
    x2d = x.reshape(N, 1)

    h = None
    deg = None
    for i, lp in enumerate(params['layers']):
        wcp = lp['Wc'].reshape(HID, HEADS, NAGG * BASES).transpose(0, 2, 1)
        wcp = wcp.reshape(HID, HEADS * NAGG * BASES)
        bcp = lp['bc'].reshape(HEADS, NAGG * BASES).T.reshape(1, -1)
        if i == 0:
            h, bases, w2 = _tcA0(x2d, emb, lp['Wb'], wcp, bcp)
        else:
            bases, w2 = _tcA(h, lp['Wb'], wcp, bcp)
        asum3, amax3, deg3 = _agg_fn()(bases, pd, cntm)
        asum = asum3.reshape(NPAD, BF)
        amax = amax3.reshape(NPAD, BF)
        if deg is None:
            deg = deg3.reshape(NPAD, 1)
        h = _tcB(w2, asum, amax, deg, lp['bias'].reshape(1, HID),
                 lp['gamma'].reshape(1, HID), lp['beta'].reshape(1, HID), h)

    mp = params['mlp']
    w3 = jnp.zeros((HID // 4, 8), jnp.float32).at[:, :1].set(mp['W3'])
    b3 = jnp.zeros((1, 8), jnp.float32).at[:, :1].set(mp['b3'].reshape(1, 1))
    y8 = _tcF(h, batch.reshape(1, N), mp['W1'], mp['b1'].reshape(1, -1),
              mp['W2'], mp['b2'].reshape(1, -1), w3, b3)
    return y8[:, :1]
